# Initial kernel scaffold; baseline (speedup 1.0000x reference)
import jax
import jax.numpy as jnp
from jax.experimental import pallas as pl


def kernel(embedding_1, embedding_2, edge_label_index):
    raise NotImplementedError("write your pallas kernel here")


# speedup vs baseline: 9.3316x; 9.3316x over previous
import functools

import jax
import jax.numpy as jnp
from jax import lax
from jax.experimental import pallas as pl
from jax.experimental.pallas import tpu as pltpu
from jax.experimental.pallas import tpu_sc as plsc

_N_EDGES = 320000
_D = 128
_NW = 32
_EW = _N_EDGES // _NW
_C = 80
_NCH = _EW // _C
_G = _C // 16


def _body(src_hbm, dst_hbm, emb1_hbm, emb2_hbm, out_hbm,
          idx1_v, idx2_v, a0, b0, a1, b1, out_v, sa0, sb0, sa1, sb1):
    nc = 2
    wid = lax.axis_index("s") * nc + lax.axis_index("c")
    base = wid * _EW

    pltpu.sync_copy(src_hbm.at[pl.ds(base, _EW)], idx1_v)
    pltpu.sync_copy(dst_hbm.at[pl.ds(base, _EW)], idx2_v)

    lane = lax.iota(jnp.int32, 16)

    def start(c, abuf, bbuf, sa, sb):
        off = c * _C
        pltpu.async_copy(emb1_hbm.at[idx1_v.at[pl.ds(off, _C)]], abuf, sa)
        pltpu.async_copy(emb2_hbm.at[idx2_v.at[pl.ds(off, _C)]], bbuf, sb)

    def wait(c, abuf, bbuf, sa, sb):
        off = c * _C
        pltpu.make_async_copy(emb1_hbm.at[idx1_v.at[pl.ds(off, _C)]], abuf, sa).wait()
        pltpu.make_async_copy(emb2_hbm.at[idx2_v.at[pl.ds(off, _C)]], bbuf, sb).wait()

    def compute(c, abuf, bbuf):
        out_off = c * _C
        for g in range(_G):
            row = lane + (g * 16)

            def dstep(t, acc):
                col = (lane + t) & (_D - 1)
                va = plsc.load_gather(abuf, [row, col])
                vb = plsc.load_gather(bbuf, [row, col])
                return acc + va * vb

            acc = lax.fori_loop(0, _D, dstep, jnp.zeros((16,), jnp.float32),
                                unroll=16)
            out_v[pl.ds(out_off + g * 16, 16)] = acc

    start(0, a0, b0, sa0, sb0)

    @pl.loop(0, (_NCH - 1) // 2)
    def _pair(i):
        c0 = 2 * i
        start(c0 + 1, a1, b1, sa1, sb1)
        wait(c0, a0, b0, sa0, sb0)
        compute(c0, a0, b0)
        start(c0 + 2, a0, b0, sa0, sb0)
        wait(c0 + 1, a1, b1, sa1, sb1)
        compute(c0 + 1, a1, b1)

    wait(_NCH - 1, a0, b0, sa0, sb0)
    compute(_NCH - 1, a0, b0)

    pltpu.sync_copy(out_v, out_hbm.at[pl.ds(base, _EW)])


@jax.jit
def _sc_dot(src, dst, emb1, emb2):
    mesh = plsc.VectorSubcoreMesh(core_axis_name="c", subcore_axis_name="s")
    return pl.kernel(
        _body,
        out_type=jax.ShapeDtypeStruct((_N_EDGES,), jnp.float32),
        mesh=mesh,
        compiler_params=pltpu.CompilerParams(needs_layout_passes=False),
        scratch_types=[
            pltpu.VMEM((_EW,), jnp.int32),
            pltpu.VMEM((_EW,), jnp.int32),
            pltpu.VMEM((_C, _D), jnp.float32),
            pltpu.VMEM((_C, _D), jnp.float32),
            pltpu.VMEM((_C, _D), jnp.float32),
            pltpu.VMEM((_C, _D), jnp.float32),
            pltpu.VMEM((_EW,), jnp.float32),
            pltpu.SemaphoreType.DMA,
            pltpu.SemaphoreType.DMA,
            pltpu.SemaphoreType.DMA,
            pltpu.SemaphoreType.DMA,
        ],
    )(src, dst, emb1, emb2)


def kernel(embedding_1, embedding_2, edge_label_index):
    src = edge_label_index[0].astype(jnp.int32)
    dst = edge_label_index[1].astype(jnp.int32)
    return _sc_dot(src, dst, embedding_1, embedding_2)
